# Initial kernel scaffold; baseline (speedup 1.0000x reference)
import jax
import jax.numpy as jnp
from jax.experimental import pallas as pl


def kernel(xyz, new_xyz):
    raise NotImplementedError("write your pallas kernel here")


# speedup vs baseline: 1.4177x; 1.4177x over previous
import math

import jax
import jax.numpy as jnp
from jax.experimental import pallas as pl

_RADIUS = 0.2
_NS = 32
_K = _NS + 1
_B = 8
_N = 8192
_S = 1024
_ST = 128
_NC = 512


def _qkernel(xyz_ref, new_ref, rnd_ref, out_ref):
    f32 = jnp.float32
    x = xyz_ref[0]
    qx = new_ref[0, :, 0:1]
    qy = new_ref[0, :, 1:2]
    qz = new_ref[0, :, 2:3]

    r2 = jnp.asarray(_RADIUS * _RADIUS, f32)
    kvec = jax.lax.broadcasted_iota(jnp.int32, (_K, 1, 1), 0).astype(f32)

    rows = jax.lax.broadcasted_iota(jnp.int32, (_NC, _NC), 0)
    cols = jax.lax.broadcasted_iota(jnp.int32, (_NC, _NC), 1)
    ltinc = (rows <= cols).astype(f32)

    def body(c, carry):
        coords, tot = carry
        xc = xyz_ref[0, :, pl.ds(c * _NC, _NC)]
        dxx = qx - xc[0:1, :]
        dyy = qy - xc[1:2, :]
        dzz = qz - xc[2:3, :]
        d2 = dxx * dxx + dyy * dyy + dzz * dzz
        maskf = (d2 < r2).astype(f32)
        within = jax.lax.dot_general(
            maskf, ltinc, (((1,), (0,)), ((), ())),
            preferred_element_type=f32)
        cexc = within + tot - maskf
        cc = jnp.where(maskf > 0, cexc, -1.0)
        sel = (cc[None, :, :] == kvec).astype(f32)
        sel2 = sel.reshape(_K * _ST, _NC)
        coords = coords + jax.lax.dot_general(
            xc, sel2, (((1,), (1,)), ((), ())),
            preferred_element_type=f32,
            precision=jax.lax.Precision.HIGHEST)
        tot = tot + within[:, _NC - 1:_NC]
        return coords, tot

    coords0 = jnp.zeros((3, _K * _ST), f32)
    tot0 = jnp.zeros((_ST, 1), f32)
    coords, tot = jax.lax.fori_loop(0, _N // _NC, body, (coords0, tot0))

    c3 = coords.reshape(3, _K, _ST)
    cx = c3[0].T
    cy = c3[1].T
    cz = c3[2].T

    cnt = jnp.minimum(tot, float(_K))
    has = tot > 0
    padx = jnp.where(has, cx[:, 0:1], x[0, 0])
    pady = jnp.where(has, cy[:, 0:1], x[1, 0])
    padz = jnp.where(has, cz[:, 0:1], x[2, 0])
    kio = jax.lax.broadcasted_iota(jnp.int32, (1, _K), 1).astype(f32)
    vld = kio < cnt
    cx = jnp.where(vld, cx, padx)
    cy = jnp.where(vld, cy, pady)
    cz = jnp.where(vld, cz, padz)

    relx = cx[:, 1:] - qx
    rely = cy[:, 1:] - qy
    relz = cz[:, 1:] - qz

    eps = 1e-06
    pn = jnp.sqrt(qx * qx + qy * qy + qz * qz)
    p1x = qx / (pn + eps)
    p1y = qy / (pn + eps)
    p1z = qz / (pn + eps)
    p1n = jnp.sqrt(p1x * p1x + p1y * p1y + p1z * p1z)
    p2x = p1x / (p1n + eps)
    p2y = p1y / (p1n + eps)
    p2z = p1z / (p1n + eps)
    col = jnp.abs(p2x) > (1.0 - 0.001)
    rx = jnp.where(col, -p2y * p2x, 1.0 - p2x * p2x)
    ry = jnp.where(col, 1.0 - p2y * p2y, -p2x * p2y)
    rz = jnp.where(col, -p2y * p2z, -p2x * p2z)
    rn = jnp.sqrt(rx * rx + ry * ry + rz * rz)
    refx = rx / (rn + eps)
    refy = ry / (rn + eps)
    refz = rz / (rn + eps)

    vert = p2x * relx + p2y * rely + p2z * relz
    pjx = relx - vert * p2x
    pjy = rely - vert * p2y
    pjz = relz - vert * p2z
    pjn = jnp.sqrt(pjx * pjx + pjy * pjy + pjz * pjz)
    inv = 1.0 / (pjn + eps)
    ux = pjx * inv
    uy = pjy * inv
    uz = pjz * inv
    close = (ux * ux + uy * uy + uz * uz) < 1e-12

    crx = refy * uz - refz * uy
    cry = refz * ux - refx * uz
    crz = refx * uy - refy * ux
    sinus = crx * p1x + cry * p1y + crz * p1z
    cosin = refx * ux + refy * uy + refz * uz
    ang = jnp.arctan2(sinus, cosin)
    ang = jnp.where(close, rnd_ref[0], ang)

    ai = ang[:, :, None]
    aj = ang[:, None, :]
    iio = jax.lax.broadcasted_iota(jnp.int32, (1, _NS, _NS), 1)
    jio = jax.lax.broadcasted_iota(jnp.int32, (1, _NS, _NS), 2)
    less = (aj < ai) | ((aj == ai) & (jio < iio))
    rank = jnp.sum(less.astype(f32), axis=2)

    rio = jax.lax.broadcasted_iota(jnp.int32, (1, 1, _NS), 2).astype(f32)
    oh = (rank[:, :, None] == rio).astype(f32)
    sx = jnp.sum(relx[:, :, None] * oh, axis=1)
    sy = jnp.sum(rely[:, :, None] * oh, axis=1)
    sz = jnp.sum(relz[:, :, None] * oh, axis=1)

    dist = jnp.sqrt(sx * sx + sy * sy + sz * sz)
    th = dist / _RADIUS * math.pi / 2.0
    ct = jnp.cos(th)
    st = jnp.sin(th)
    dinv = 1.0 / (dist + eps)
    qvx = st * sx * dinv
    qvy = st * sy * dinv
    qvz = st * sz * dinv

    chans = (ct, qvx, qvy, qvz)
    for ci in range(4):
        v = chans[ci]
        for m in range(8):
            if m == 0:
                rolled = v
            else:
                rolled = jnp.concatenate([v[:, m:], v[:, :m]], axis=1)
            out_ref[0, 8 * ci + m] = rolled


def kernel(xyz, new_xyz):
    xyz_t = jnp.transpose(xyz, (0, 2, 1))
    rnd = (jax.random.uniform(jax.random.key(42), (_B, 1, _S, _NS),
                              dtype=jnp.float32) - 0.5) * math.pi * 2
    rnd = rnd.reshape(_B, _S, _NS)
    grid = (_B, _S // _ST)
    return pl.pallas_call(
        _qkernel,
        grid=grid,
        in_specs=[
            pl.BlockSpec((1, 3, _N), lambda b, t: (b, 0, 0)),
            pl.BlockSpec((1, _ST, 3), lambda b, t: (b, t, 0)),
            pl.BlockSpec((1, _ST, _NS), lambda b, t: (b, t, 0)),
        ],
        out_specs=pl.BlockSpec((1, 32, _ST, _NS), lambda b, t: (b, 0, t, 0)),
        out_shape=jax.ShapeDtypeStruct((_B, 32, _S, _NS), jnp.float32),
    )(xyz_t, new_xyz, rnd)


# speedup vs baseline: 4.4918x; 3.1685x over previous
import math

import jax
import jax.numpy as jnp
from jax.experimental import pallas as pl

_RADIUS = 0.2
_NS = 32
_K = _NS + 1
_B = 8
_N = 8192
_S = 1024
_ST = 128
_NC = 512


def _qkernel(xyz_ref, xyzr_ref, new_ref, rnd_ref, out_ref):
    f32 = jnp.float32
    bf16 = jnp.bfloat16
    x = xyz_ref[0]
    qx = new_ref[0, :, 0:1]
    qy = new_ref[0, :, 1:2]
    qz = new_ref[0, :, 2:3]

    r2 = jnp.asarray(_RADIUS * _RADIUS, f32)
    kvec = jax.lax.broadcasted_iota(jnp.int32, (_K, 1, 1), 0).astype(f32)

    rows = jax.lax.broadcasted_iota(jnp.int32, (_NC, _NC), 0)
    cols = jax.lax.broadcasted_iota(jnp.int32, (_NC, _NC), 1)
    ltinc = (rows <= cols).astype(bf16)

    def body(c, carry):
        coords, tot = carry
        xc = xyz_ref[0, :, pl.ds(c * _NC, _NC)]
        xcr = xyzr_ref[0, pl.ds(c * _NC, _NC), :]
        xh = xcr.astype(bf16)
        xm = (xcr - xh.astype(f32)).astype(bf16)
        xl = (xcr - xh.astype(f32) - xm.astype(f32)).astype(bf16)
        dxx = qx - xc[0:1, :]
        dyy = qy - xc[1:2, :]
        dzz = qz - xc[2:3, :]
        d2 = dxx * dxx + dyy * dyy + dzz * dzz
        mb = d2 < r2
        maskf = mb.astype(f32)
        within = jax.lax.dot_general(
            mb.astype(bf16), ltinc, (((1,), (0,)), ((), ())),
            preferred_element_type=f32)
        cexc = within + tot - maskf
        cc = jnp.where(mb, cexc, -1.0)
        sel = (cc[None, :, :] == kvec).astype(bf16)
        sel2 = sel.reshape(_K * _ST, _NC)
        dn = (((1,), (0,)), ((), ()))
        coords = coords + (
            jax.lax.dot_general(sel2, xh, dn, preferred_element_type=f32)
            + jax.lax.dot_general(sel2, xm, dn, preferred_element_type=f32)
            + jax.lax.dot_general(sel2, xl, dn, preferred_element_type=f32))
        tot = tot + within[:, _NC - 1:_NC]
        return coords, tot

    coords0 = jnp.zeros((_K * _ST, 3), f32)
    tot0 = jnp.zeros((_ST, 1), f32)
    coords, tot = jax.lax.fori_loop(0, _N // _NC, body, (coords0, tot0))

    c3 = coords.reshape(_K, _ST, 3)
    cx = c3[:, :, 0].T
    cy = c3[:, :, 1].T
    cz = c3[:, :, 2].T

    cnt = jnp.minimum(tot, float(_K))
    has = tot > 0
    padx = jnp.where(has, cx[:, 0:1], x[0, 0])
    pady = jnp.where(has, cy[:, 0:1], x[1, 0])
    padz = jnp.where(has, cz[:, 0:1], x[2, 0])
    kio = jax.lax.broadcasted_iota(jnp.int32, (1, _K), 1).astype(f32)
    vld = kio < cnt
    cx = jnp.where(vld, cx, padx)
    cy = jnp.where(vld, cy, pady)
    cz = jnp.where(vld, cz, padz)

    relx = cx[:, 1:] - qx
    rely = cy[:, 1:] - qy
    relz = cz[:, 1:] - qz

    eps = 1e-06
    pn = jnp.sqrt(qx * qx + qy * qy + qz * qz)
    p1x = qx / (pn + eps)
    p1y = qy / (pn + eps)
    p1z = qz / (pn + eps)
    p1n = jnp.sqrt(p1x * p1x + p1y * p1y + p1z * p1z)
    p2x = p1x / (p1n + eps)
    p2y = p1y / (p1n + eps)
    p2z = p1z / (p1n + eps)
    col = jnp.abs(p2x) > (1.0 - 0.001)
    rx = jnp.where(col, -p2y * p2x, 1.0 - p2x * p2x)
    ry = jnp.where(col, 1.0 - p2y * p2y, -p2x * p2y)
    rz = jnp.where(col, -p2y * p2z, -p2x * p2z)
    rn = jnp.sqrt(rx * rx + ry * ry + rz * rz)
    refx = rx / (rn + eps)
    refy = ry / (rn + eps)
    refz = rz / (rn + eps)

    vert = p2x * relx + p2y * rely + p2z * relz
    pjx = relx - vert * p2x
    pjy = rely - vert * p2y
    pjz = relz - vert * p2z
    pjn = jnp.sqrt(pjx * pjx + pjy * pjy + pjz * pjz)
    inv = 1.0 / (pjn + eps)
    ux = pjx * inv
    uy = pjy * inv
    uz = pjz * inv
    close = (ux * ux + uy * uy + uz * uz) < 1e-12

    crx = refy * uz - refz * uy
    cry = refz * ux - refx * uz
    crz = refx * uy - refy * ux
    sinus = crx * p1x + cry * p1y + crz * p1z
    cosin = refx * ux + refy * uy + refz * uz
    ang = jnp.arctan2(sinus, cosin)
    ang = jnp.where(close, rnd_ref[0], ang)

    ai = ang[:, :, None]
    aj = ang[:, None, :]
    iio = jax.lax.broadcasted_iota(jnp.int32, (1, _NS, _NS), 1)
    jio = jax.lax.broadcasted_iota(jnp.int32, (1, _NS, _NS), 2)
    less = (aj < ai) | ((aj == ai) & (jio < iio))
    rank = jnp.sum(less.astype(f32), axis=2)

    rio = jax.lax.broadcasted_iota(jnp.int32, (1, 1, _NS), 2).astype(f32)
    oh = (rank[:, :, None] == rio).astype(f32)
    sx = jnp.sum(relx[:, :, None] * oh, axis=1)
    sy = jnp.sum(rely[:, :, None] * oh, axis=1)
    sz = jnp.sum(relz[:, :, None] * oh, axis=1)

    dist = jnp.sqrt(sx * sx + sy * sy + sz * sz)
    th = dist / _RADIUS * math.pi / 2.0
    ct = jnp.cos(th)
    st = jnp.sin(th)
    dinv = 1.0 / (dist + eps)
    qvx = st * sx * dinv
    qvy = st * sy * dinv
    qvz = st * sz * dinv

    chans = (ct, qvx, qvy, qvz)
    for ci in range(4):
        v = chans[ci]
        for m in range(8):
            if m == 0:
                rolled = v
            else:
                rolled = jnp.concatenate([v[:, m:], v[:, :m]], axis=1)
            out_ref[0, 8 * ci + m] = rolled


def kernel(xyz, new_xyz):
    xyz_t = jnp.transpose(xyz, (0, 2, 1))
    rnd = (jax.random.uniform(jax.random.key(42), (_B, 1, _S, _NS),
                              dtype=jnp.float32) - 0.5) * math.pi * 2
    rnd = rnd.reshape(_B, _S, _NS)
    grid = (_B, _S // _ST)
    return pl.pallas_call(
        _qkernel,
        grid=grid,
        in_specs=[
            pl.BlockSpec((1, 3, _N), lambda b, t: (b, 0, 0)),
            pl.BlockSpec((1, _N, 3), lambda b, t: (b, 0, 0)),
            pl.BlockSpec((1, _ST, 3), lambda b, t: (b, t, 0)),
            pl.BlockSpec((1, _ST, _NS), lambda b, t: (b, t, 0)),
        ],
        out_specs=pl.BlockSpec((1, 32, _ST, _NS), lambda b, t: (b, 0, t, 0)),
        out_shape=jax.ShapeDtypeStruct((_B, 32, _S, _NS), jnp.float32),
    )(xyz_t, xyz, new_xyz, rnd)


# speedup vs baseline: 7.5487x; 1.6805x over previous
import math

import jax
import jax.numpy as jnp
from jax.experimental import pallas as pl

_RADIUS = 0.2
_NS = 32
_K = _NS + 1
_B = 8
_N = 8192
_S = 1024
_ST = 128
_NC = 512


def _qkernel(xyz_ref, xyzr_ref, new_ref, rnd_ref, out_ref):
    f32 = jnp.float32
    bf16 = jnp.bfloat16
    x = xyz_ref[0]
    qx = new_ref[0, :, 0:1]
    qy = new_ref[0, :, 1:2]
    qz = new_ref[0, :, 2:3]

    r2 = jnp.asarray(_RADIUS * _RADIUS, f32)
    kvec = jax.lax.broadcasted_iota(jnp.int32, (_K, 1, 1), 0).astype(f32)

    rows = jax.lax.broadcasted_iota(jnp.int32, (_NC, _NC), 0)
    cols = jax.lax.broadcasted_iota(jnp.int32, (_NC, _NC), 1)
    ltinc = (rows <= cols).astype(bf16)

    def cond(carry):
        c, coords, tot = carry
        return jnp.logical_and(c < _N // _NC, jnp.min(tot) < float(_K))

    def body(carry):
        c, coords, tot = carry
        xc = xyz_ref[0, :, pl.ds(c * _NC, _NC)]
        xcr = xyzr_ref[0, pl.ds(c * _NC, _NC), :]
        xh = xcr.astype(bf16)
        xm = (xcr - xh.astype(f32)).astype(bf16)
        xl = (xcr - xh.astype(f32) - xm.astype(f32)).astype(bf16)
        dxx = qx - xc[0:1, :]
        dyy = qy - xc[1:2, :]
        dzz = qz - xc[2:3, :]
        d2 = dxx * dxx + dyy * dyy + dzz * dzz
        mb = d2 < r2
        maskf = mb.astype(f32)
        within = jax.lax.dot_general(
            mb.astype(bf16), ltinc, (((1,), (0,)), ((), ())),
            preferred_element_type=f32)
        cexc = within + tot - maskf
        cc = jnp.where(mb, cexc, -1.0)
        sel = (cc[None, :, :] == kvec).astype(bf16)
        sel2 = sel.reshape(_K * _ST, _NC)
        dn = (((1,), (0,)), ((), ()))
        coords = coords + (
            jax.lax.dot_general(sel2, xh, dn, preferred_element_type=f32)
            + jax.lax.dot_general(sel2, xm, dn, preferred_element_type=f32)
            + jax.lax.dot_general(sel2, xl, dn, preferred_element_type=f32))
        tot = tot + within[:, _NC - 1:_NC]
        return c + 1, coords, tot

    coords0 = jnp.zeros((_K * _ST, 3), f32)
    tot0 = jnp.zeros((_ST, 1), f32)
    _, coords, tot = jax.lax.while_loop(
        cond, body, (jnp.int32(0), coords0, tot0))

    c3 = coords.reshape(_K, _ST, 3)
    cx = c3[:, :, 0].T
    cy = c3[:, :, 1].T
    cz = c3[:, :, 2].T

    cnt = jnp.minimum(tot, float(_K))
    has = tot > 0
    padx = jnp.where(has, cx[:, 0:1], x[0, 0])
    pady = jnp.where(has, cy[:, 0:1], x[1, 0])
    padz = jnp.where(has, cz[:, 0:1], x[2, 0])
    kio = jax.lax.broadcasted_iota(jnp.int32, (1, _K), 1).astype(f32)
    vld = kio < cnt
    cx = jnp.where(vld, cx, padx)
    cy = jnp.where(vld, cy, pady)
    cz = jnp.where(vld, cz, padz)

    relx = cx[:, 1:] - qx
    rely = cy[:, 1:] - qy
    relz = cz[:, 1:] - qz

    eps = 1e-06
    pn = jnp.sqrt(qx * qx + qy * qy + qz * qz)
    p1x = qx / (pn + eps)
    p1y = qy / (pn + eps)
    p1z = qz / (pn + eps)
    p1n = jnp.sqrt(p1x * p1x + p1y * p1y + p1z * p1z)
    p2x = p1x / (p1n + eps)
    p2y = p1y / (p1n + eps)
    p2z = p1z / (p1n + eps)
    col = jnp.abs(p2x) > (1.0 - 0.001)
    rx = jnp.where(col, -p2y * p2x, 1.0 - p2x * p2x)
    ry = jnp.where(col, 1.0 - p2y * p2y, -p2x * p2y)
    rz = jnp.where(col, -p2y * p2z, -p2x * p2z)
    rn = jnp.sqrt(rx * rx + ry * ry + rz * rz)
    refx = rx / (rn + eps)
    refy = ry / (rn + eps)
    refz = rz / (rn + eps)

    vert = p2x * relx + p2y * rely + p2z * relz
    pjx = relx - vert * p2x
    pjy = rely - vert * p2y
    pjz = relz - vert * p2z
    pjn = jnp.sqrt(pjx * pjx + pjy * pjy + pjz * pjz)
    inv = 1.0 / (pjn + eps)
    ux = pjx * inv
    uy = pjy * inv
    uz = pjz * inv
    close = (ux * ux + uy * uy + uz * uz) < 1e-12

    crx = refy * uz - refz * uy
    cry = refz * ux - refx * uz
    crz = refx * uy - refy * ux
    sinus = crx * p1x + cry * p1y + crz * p1z
    cosin = refx * ux + refy * uy + refz * uz
    ang = jnp.arctan2(sinus, cosin)
    ang = jnp.where(close, rnd_ref[0], ang)

    ai = ang[:, :, None]
    aj = ang[:, None, :]
    iio = jax.lax.broadcasted_iota(jnp.int32, (1, _NS, _NS), 1)
    jio = jax.lax.broadcasted_iota(jnp.int32, (1, _NS, _NS), 2)
    less = (aj < ai) | ((aj == ai) & (jio < iio))
    rank = jnp.sum(less.astype(f32), axis=2)

    rio = jax.lax.broadcasted_iota(jnp.int32, (1, 1, _NS), 2).astype(f32)
    oh = (rank[:, :, None] == rio).astype(f32)
    sx = jnp.sum(relx[:, :, None] * oh, axis=1)
    sy = jnp.sum(rely[:, :, None] * oh, axis=1)
    sz = jnp.sum(relz[:, :, None] * oh, axis=1)

    dist = jnp.sqrt(sx * sx + sy * sy + sz * sz)
    th = dist / _RADIUS * math.pi / 2.0
    ct = jnp.cos(th)
    st = jnp.sin(th)
    dinv = 1.0 / (dist + eps)
    qvx = st * sx * dinv
    qvy = st * sy * dinv
    qvz = st * sz * dinv

    chans = (ct, qvx, qvy, qvz)
    for ci in range(4):
        v = chans[ci]
        for m in range(8):
            if m == 0:
                rolled = v
            else:
                rolled = jnp.concatenate([v[:, m:], v[:, :m]], axis=1)
            out_ref[0, 8 * ci + m] = rolled


def kernel(xyz, new_xyz):
    xyz_t = jnp.transpose(xyz, (0, 2, 1))
    rnd = (jax.random.uniform(jax.random.key(42), (_B, 1, _S, _NS),
                              dtype=jnp.float32) - 0.5) * math.pi * 2
    rnd = rnd.reshape(_B, _S, _NS)
    grid = (_B, _S // _ST)
    return pl.pallas_call(
        _qkernel,
        grid=grid,
        in_specs=[
            pl.BlockSpec((1, 3, _N), lambda b, t: (b, 0, 0)),
            pl.BlockSpec((1, _N, 3), lambda b, t: (b, 0, 0)),
            pl.BlockSpec((1, _ST, 3), lambda b, t: (b, t, 0)),
            pl.BlockSpec((1, _ST, _NS), lambda b, t: (b, t, 0)),
        ],
        out_specs=pl.BlockSpec((1, 32, _ST, _NS), lambda b, t: (b, 0, t, 0)),
        out_shape=jax.ShapeDtypeStruct((_B, 32, _S, _NS), jnp.float32),
    )(xyz_t, xyz, new_xyz, rnd)


# speedup vs baseline: 12.7464x; 1.6886x over previous
import math

import jax
import jax.numpy as jnp
from jax.experimental import pallas as pl
from jax.experimental.pallas import tpu as pltpu

_RADIUS = 0.2
_NS = 32
_K = _NS + 1
_B = 8
_N = 8192
_S = 1024
_ST = 128
_NC = 512


def _qkernel(xyz_ref, xyzr_ref, new_ref, rnd_ref, out_ref):
    f32 = jnp.float32
    bf16 = jnp.bfloat16
    x = xyz_ref[0]
    qx = new_ref[0, :, 0:1]
    qy = new_ref[0, :, 1:2]
    qz = new_ref[0, :, 2:3]

    r2 = jnp.asarray(_RADIUS * _RADIUS, f32)
    kvec = jax.lax.broadcasted_iota(jnp.int32, (_K, 1, 1), 0).astype(f32)

    rows = jax.lax.broadcasted_iota(jnp.int32, (_NC, _NC), 0)
    cols = jax.lax.broadcasted_iota(jnp.int32, (_NC, _NC), 1)
    ltinc = (rows <= cols).astype(bf16)

    def cond(carry):
        c, coords, tot = carry
        return jnp.logical_and(c < _N // _NC, jnp.min(tot) < float(_K))

    def body(carry):
        c, coords, tot = carry
        xc = xyz_ref[0, :, pl.ds(c * _NC, _NC)]
        xcr = xyzr_ref[0, pl.ds(c * _NC, _NC), :]
        xh = xcr.astype(bf16)
        xm = (xcr - xh.astype(f32)).astype(bf16)
        xl = (xcr - xh.astype(f32) - xm.astype(f32)).astype(bf16)
        xs = jnp.concatenate([xh, xm, xl], axis=1)
        dxx = qx - xc[0:1, :]
        dyy = qy - xc[1:2, :]
        dzz = qz - xc[2:3, :]
        d2 = dxx * dxx + dyy * dyy + dzz * dzz
        mb = d2 < r2
        maskf = mb.astype(f32)
        within = jax.lax.dot_general(
            mb.astype(bf16), ltinc, (((1,), (0,)), ((), ())),
            preferred_element_type=f32)
        cexc = within + tot - maskf
        cc = jnp.where(mb, cexc, -1.0)
        sel = (cc[None, :, :] == kvec).astype(bf16)
        sel2 = sel.reshape(_K * _ST, _NC)
        dn = (((1,), (0,)), ((), ()))
        cs = jax.lax.dot_general(sel2, xs, dn,
                                 preferred_element_type=f32)
        coords = coords + (cs[:, 0:3] + cs[:, 3:6] + cs[:, 6:9])
        tot = tot + within[:, _NC - 1:_NC]
        return c + 1, coords, tot

    coords0 = jnp.zeros((_K * _ST, 3), f32)
    tot0 = jnp.zeros((_ST, 1), f32)
    _, coords, tot = jax.lax.while_loop(
        cond, body, (jnp.int32(0), coords0, tot0))

    c3 = coords.reshape(_K, _ST, 3)
    cx = c3[:, :, 0].T
    cy = c3[:, :, 1].T
    cz = c3[:, :, 2].T

    cnt = jnp.minimum(tot, float(_K))
    has = tot > 0
    padx = jnp.where(has, cx[:, 0:1], x[0, 0])
    pady = jnp.where(has, cy[:, 0:1], x[1, 0])
    padz = jnp.where(has, cz[:, 0:1], x[2, 0])
    kio = jax.lax.broadcasted_iota(jnp.int32, (1, _K), 1).astype(f32)
    vld = kio < cnt
    cx = jnp.where(vld, cx, padx)
    cy = jnp.where(vld, cy, pady)
    cz = jnp.where(vld, cz, padz)

    relx = cx[:, 1:] - qx
    rely = cy[:, 1:] - qy
    relz = cz[:, 1:] - qz

    eps = 1e-06
    pn = jnp.sqrt(qx * qx + qy * qy + qz * qz)
    p1x = qx / (pn + eps)
    p1y = qy / (pn + eps)
    p1z = qz / (pn + eps)
    p1n = jnp.sqrt(p1x * p1x + p1y * p1y + p1z * p1z)
    p2x = p1x / (p1n + eps)
    p2y = p1y / (p1n + eps)
    p2z = p1z / (p1n + eps)
    col = jnp.abs(p2x) > (1.0 - 0.001)
    rx = jnp.where(col, -p2y * p2x, 1.0 - p2x * p2x)
    ry = jnp.where(col, 1.0 - p2y * p2y, -p2x * p2y)
    rz = jnp.where(col, -p2y * p2z, -p2x * p2z)
    rn = jnp.sqrt(rx * rx + ry * ry + rz * rz)
    refx = rx / (rn + eps)
    refy = ry / (rn + eps)
    refz = rz / (rn + eps)

    vert = p2x * relx + p2y * rely + p2z * relz
    pjx = relx - vert * p2x
    pjy = rely - vert * p2y
    pjz = relz - vert * p2z
    pjn = jnp.sqrt(pjx * pjx + pjy * pjy + pjz * pjz)
    inv = 1.0 / (pjn + eps)
    ux = pjx * inv
    uy = pjy * inv
    uz = pjz * inv
    close = (ux * ux + uy * uy + uz * uz) < 1e-12

    crx = refy * uz - refz * uy
    cry = refz * ux - refx * uz
    crz = refx * uy - refy * ux
    sinus = crx * p1x + cry * p1y + crz * p1z
    cosin = refx * ux + refy * uy + refz * uz
    ang = jnp.arctan2(sinus, cosin)
    ang = jnp.where(close, rnd_ref[0], ang)

    ai = ang[:, :, None]
    aj = ang[:, None, :]
    iio = jax.lax.broadcasted_iota(jnp.int32, (1, _NS, _NS), 1)
    jio = jax.lax.broadcasted_iota(jnp.int32, (1, _NS, _NS), 2)
    less = (aj < ai) | ((aj == ai) & (jio < iio))
    rank = jnp.sum(less.astype(f32), axis=2)

    rio = jax.lax.broadcasted_iota(jnp.int32, (1, 1, _NS), 2).astype(f32)
    oh = (rank[:, :, None] == rio).astype(f32)
    sx = jnp.sum(relx[:, :, None] * oh, axis=1)
    sy = jnp.sum(rely[:, :, None] * oh, axis=1)
    sz = jnp.sum(relz[:, :, None] * oh, axis=1)

    dist = jnp.sqrt(sx * sx + sy * sy + sz * sz)
    th = dist / _RADIUS * math.pi / 2.0

    def _polytrig(tt):
        t2 = tt * tt
        c = 1.0 + t2 * (-1.0 / 2 + t2 * (1.0 / 24 + t2 * (-1.0 / 720
            + t2 * (1.0 / 40320 + t2 * (-1.0 / 3628800 + t2 / 479001600)))))
        s = tt * (1.0 + t2 * (-1.0 / 6 + t2 * (1.0 / 120 + t2 * (-1.0 / 5040
            + t2 * (1.0 / 362880 + t2 * (-1.0 / 39916800))))))
        return c, s

    def _exacttrig(tt):
        return jnp.cos(tt), jnp.sin(tt)

    ct, st = jax.lax.cond(jnp.min(tot) > 0, _polytrig, _exacttrig, th)
    dinv = 1.0 / (dist + eps)
    qvx = st * sx * dinv
    qvy = st * sy * dinv
    qvz = st * sz * dinv

    chans = (ct, qvx, qvy, qvz)
    for ci in range(4):
        v = chans[ci]
        for m in range(8):
            if m == 0:
                rolled = v
            else:
                rolled = jnp.concatenate([v[:, m:], v[:, :m]], axis=1)
            out_ref[0, 8 * ci + m] = rolled


def kernel(xyz, new_xyz):
    xyz_t = jnp.transpose(xyz, (0, 2, 1))
    rnd = (jax.random.uniform(jax.random.key(42), (_B, 1, _S, _NS),
                              dtype=jnp.float32) - 0.5) * math.pi * 2
    rnd = rnd.reshape(_B, _S, _NS)
    grid = (_B, _S // _ST)
    return pl.pallas_call(
        _qkernel,
        grid=grid,
        in_specs=[
            pl.BlockSpec((1, 3, _N), lambda b, t: (b, 0, 0)),
            pl.BlockSpec((1, _N, 3), lambda b, t: (b, 0, 0)),
            pl.BlockSpec((1, _ST, 3), lambda b, t: (b, t, 0)),
            pl.BlockSpec((1, _ST, _NS), lambda b, t: (b, t, 0)),
        ],
        out_specs=pl.BlockSpec((1, 32, _ST, _NS), lambda b, t: (b, 0, t, 0)),
        out_shape=jax.ShapeDtypeStruct((_B, 32, _S, _NS), jnp.float32),
        compiler_params=pltpu.CompilerParams(
            dimension_semantics=("parallel", "parallel")),
    )(xyz_t, xyz, new_xyz, rnd)
